# Initial kernel scaffold; baseline (speedup 1.0000x reference)
import jax
import jax.numpy as jnp
from jax.experimental import pallas as pl


def kernel(X):
    raise NotImplementedError("write your pallas kernel here")


# speedup vs baseline: 1.5715x; 1.5715x over previous
import functools

import jax
import jax.numpy as jnp
from jax.experimental import pallas as pl
from jax.experimental.pallas import tpu as pltpu

_N = 32768
_K = 512
_F = 64
_CH = 1024
_GRID = _N // _CH


def _labels_from_dist(x, c):
    a2 = jnp.sum(x * x, axis=1, keepdims=True)
    b2 = jnp.sum(c * c, axis=1)
    prod = jax.lax.dot_general(
        x, c, (((1,), (1,)), ((), ())),
        preferred_element_type=jnp.float32,
        precision=jax.lax.Precision.DEFAULT,
    )
    d2 = a2 + b2[None, :] - 2.0 * prod
    d = jnp.sqrt(jnp.maximum(d2, 0.0))
    key = jnp.where(jnp.isnan(d), -jnp.inf, d)
    m = jnp.min(key, axis=1, keepdims=True)
    cols = jax.lax.broadcasted_iota(jnp.int32, key.shape, 1)
    lab = jnp.min(jnp.where(key == m, cols, _K), axis=1)
    return lab, cols


def _iter_kernel(x_ref, sums_ref, counts_ref,
                 lab_ref, osums_ref, ocounts_ref):
    c = sums_ref[...] / counts_ref[...]
    x = x_ref[...]
    lab, cols = _labels_from_dist(x, c)
    lab_ref[...] = lab[:, None]
    oh = (lab[:, None] == cols).astype(jnp.float32)
    ps = jax.lax.dot_general(
        oh, x, (((0,), (0,)), ((), ())),
        preferred_element_type=jnp.float32,
        precision=jax.lax.Precision.HIGHEST,
    )
    pc = jnp.sum(oh, axis=0)[:, None]

    @pl.when(pl.program_id(0) == 0)
    def _init():
        osums_ref[...] = jnp.zeros_like(osums_ref)
        ocounts_ref[...] = jnp.zeros_like(ocounts_ref)

    osums_ref[...] += ps
    ocounts_ref[...] += pc


def _assign_kernel(x_ref, sums_ref, counts_ref, lab_ref):
    c = sums_ref[...] / counts_ref[...]
    lab, _ = _labels_from_dist(x_ref[...], c)
    lab_ref[...] = lab[:, None]


_full = pl.BlockSpec((_K, _F), lambda i: (0, 0))
_fullc = pl.BlockSpec((_K, 1), lambda i: (0, 0))
_xspec = pl.BlockSpec((_CH, _F), lambda i: (i, 0))
_lspec = pl.BlockSpec((_CH, 1), lambda i: (i, 0))


_iter_call = pl.pallas_call(
    _iter_kernel,
    grid=(_GRID,),
    in_specs=[_xspec, _full, _fullc],
    out_specs=[_lspec, _full, _fullc],
    out_shape=[
        jax.ShapeDtypeStruct((_N, 1), jnp.int32),
        jax.ShapeDtypeStruct((_K, _F), jnp.float32),
        jax.ShapeDtypeStruct((_K, 1), jnp.float32),
    ],
    compiler_params=pltpu.CompilerParams(
        dimension_semantics=("arbitrary",)),
)

_assign_call = pl.pallas_call(
    _assign_kernel,
    grid=(_GRID,),
    in_specs=[_xspec, _full, _fullc],
    out_specs=_lspec,
    out_shape=jax.ShapeDtypeStruct((_N, 1), jnp.int32),
    compiler_params=pltpu.CompilerParams(
        dimension_semantics=("arbitrary",)),
)


def kernel(X):
    sums = X[:_K, :]
    counts = jnp.ones((_K, 1), dtype=jnp.float32)
    for _ in range(3):
        _lab, sums, counts = _iter_call(X, sums, counts)
    labels = _assign_call(X, sums, counts)
    return labels.reshape(_N).astype(jnp.int32)
